# initial kernel scaffold (unmeasured)
import jax
import jax.numpy as jnp
from jax import lax
from jax.experimental import pallas as pl
from jax.experimental.pallas import tpu as pltpu

N_Y = 4


def kernel(Q, K, V):
    b, s, h, d = Q.shape
    bh = b * h
    scale = d ** -0.5

    Qt = Q.transpose(0, 2, 1, 3).reshape(bh, s, d) * scale
    Kt = K.transpose(0, 2, 1, 3).reshape(bh, s, d)
    Vt = V.transpose(0, 2, 1, 3).reshape(bh, s, d)

    def body(q_ref, k_ref, v_ref, out_ref, kfull, vfull, send_sems, recv_sems):
        my_x = lax.axis_index("x")
        my_y = lax.axis_index("y")
        my_z = lax.axis_index("z")
        left = (my_y - 1) % N_Y
        right = (my_y + 1) % N_Y

        barrier = pltpu.get_barrier_semaphore()
        for nbr in (left, right):
            pl.semaphore_signal(
                barrier, inc=1,
                device_id=(my_x, nbr, my_z),
                device_id_type=pl.DeviceIdType.MESH,
            )
        pl.semaphore_wait(barrier, 2)

        kfull[my_y] = k_ref[...]
        vfull[my_y] = v_ref[...]

        for t in range(N_Y - 1):
            src = (my_y - t) % N_Y
            k_rdma = pltpu.make_async_remote_copy(
                src_ref=kfull.at[src],
                dst_ref=kfull.at[src],
                send_sem=send_sems.at[0, t],
                recv_sem=recv_sems.at[0, t],
                device_id=(my_x, right, my_z),
                device_id_type=pl.DeviceIdType.MESH,
            )
            v_rdma = pltpu.make_async_remote_copy(
                src_ref=vfull.at[src],
                dst_ref=vfull.at[src],
                send_sem=send_sems.at[1, t],
                recv_sem=recv_sems.at[1, t],
                device_id=(my_x, right, my_z),
                device_id_type=pl.DeviceIdType.MESH,
            )
            k_rdma.start()
            v_rdma.start()
            k_rdma.wait()
            v_rdma.wait()

        def head_body(i, carry):
            q = q_ref[i]
            s_blocks = []
            for c in range(N_Y):
                s_blocks.append(
                    lax.dot_general(
                        q, kfull[c, i],
                        (((1,), (1,)), ((), ())),
                        preferred_element_type=jnp.float32,
                    )
                )
            S = jnp.concatenate(s_blocks, axis=1)
            m = jnp.max(S, axis=1, keepdims=True)
            P = jnp.exp(S - m)
            denom = jnp.sum(P, axis=1, keepdims=True)
            acc = jnp.zeros((s, d), jnp.float32)
            for c in range(N_Y):
                acc = acc + lax.dot_general(
                    P[:, c * s:(c + 1) * s], vfull[c, i],
                    (((1,), (0,)), ((), ())),
                    preferred_element_type=jnp.float32,
                )
            out_ref[i] = acc / denom
            return carry

        lax.fori_loop(0, bh, head_body, 0)

    out = pl.pallas_call(
        body,
        out_shape=jax.ShapeDtypeStruct((bh, s, d), jnp.float32),
        in_specs=[pl.BlockSpec(memory_space=pltpu.VMEM)] * 3,
        out_specs=pl.BlockSpec(memory_space=pltpu.VMEM),
        scratch_shapes=[
            pltpu.VMEM((N_Y, bh, s, d), jnp.float32),
            pltpu.VMEM((N_Y, bh, s, d), jnp.float32),
            pltpu.SemaphoreType.DMA((2, N_Y - 1)),
            pltpu.SemaphoreType.DMA((2, N_Y - 1)),
        ],
        compiler_params=pltpu.CompilerParams(collective_id=0),
    )(Qt, Kt, Vt)

    return out.reshape(b, h, s, d).transpose(0, 2, 1, 3)


# baseline (device time: 339265 ns/iter reference)
import jax
import jax.numpy as jnp
from jax import lax
from jax.experimental import pallas as pl
from jax.experimental.pallas import tpu as pltpu

N_Y = 4


def kernel(Q, K, V):
    b, s, h, d = Q.shape
    bh = b * h
    scale = d ** -0.5

    Qt = Q.transpose(0, 2, 3, 1).reshape(bh, d, s) * scale
    Kt = K.transpose(0, 2, 3, 1).reshape(bh, d, s)
    Vt = V.transpose(0, 2, 3, 1).reshape(bh, d, s)

    def body(q_ref, k_ref, v_ref, out_ref, kfull, vfull, send_sems, recv_sems):
        my_x = lax.axis_index("x")
        my_y = lax.axis_index("y")
        my_z = lax.axis_index("z")
        left = (my_y - 1) % N_Y
        right = (my_y + 1) % N_Y

        barrier = pltpu.get_barrier_semaphore()
        for nbr in (left, right):
            pl.semaphore_signal(
                barrier, inc=1,
                device_id=(my_x, nbr, my_z),
                device_id_type=pl.DeviceIdType.MESH,
            )
        pl.semaphore_wait(barrier, 2)

        for t in range(N_Y - 1):
            k_src = k_ref if t == 0 else kfull.at[t - 1]
            v_src = v_ref if t == 0 else vfull.at[t - 1]
            k_rdma = pltpu.make_async_remote_copy(
                src_ref=k_src,
                dst_ref=kfull.at[t],
                send_sem=send_sems.at[0, t],
                recv_sem=recv_sems.at[0, t],
                device_id=(my_x, right, my_z),
                device_id_type=pl.DeviceIdType.MESH,
            )
            v_rdma = pltpu.make_async_remote_copy(
                src_ref=v_src,
                dst_ref=vfull.at[t],
                send_sem=send_sems.at[1, t],
                recv_sem=recv_sems.at[1, t],
                device_id=(my_x, right, my_z),
                device_id_type=pl.DeviceIdType.MESH,
            )
            k_rdma.start()
            v_rdma.start()
            k_rdma.wait()
            v_rdma.wait()

        def head_body(i, carry):
            q = q_ref[i]
            k_chunks = [k_ref] + [kfull.at[t] for t in range(N_Y - 1)]
            v_chunks = [v_ref] + [vfull.at[t] for t in range(N_Y - 1)]
            s_blocks = []
            for kc in k_chunks:
                s_blocks.append(
                    lax.dot_general(
                        q, kc[i],
                        (((0,), (0,)), ((), ())),
                        preferred_element_type=jnp.float32,
                    )
                )
            S = jnp.concatenate(s_blocks, axis=1)
            m = jnp.max(S, axis=1, keepdims=True)
            P = jnp.exp(S - m)
            denom = jnp.sum(P, axis=1)[None, :]
            acc = jnp.zeros((d, s), jnp.float32)
            for c, vc in enumerate(v_chunks):
                acc = acc + lax.dot_general(
                    vc[i], P[:, c * s:(c + 1) * s],
                    (((1,), (1,)), ((), ())),
                    preferred_element_type=jnp.float32,
                )
            out_ref[i] = acc / denom
            return carry

        lax.fori_loop(0, bh, head_body, 0)

    out = pl.pallas_call(
        body,
        out_shape=jax.ShapeDtypeStruct((bh, d, s), jnp.float32),
        in_specs=[pl.BlockSpec(memory_space=pltpu.VMEM)] * 3,
        out_specs=pl.BlockSpec(memory_space=pltpu.VMEM),
        scratch_shapes=[
            pltpu.VMEM((N_Y - 1, bh, d, s), jnp.float32),
            pltpu.VMEM((N_Y - 1, bh, d, s), jnp.float32),
            pltpu.SemaphoreType.DMA((2, N_Y - 1)),
            pltpu.SemaphoreType.DMA((2, N_Y - 1)),
        ],
        compiler_params=pltpu.CompilerParams(collective_id=0),
    )(Qt, Kt, Vt)

    return out.reshape(b, h, d, s).transpose(0, 3, 1, 2)


# device time: 337326 ns/iter; 1.0057x vs baseline; 1.0057x over previous
import jax
import jax.numpy as jnp
from jax import lax
from jax.experimental import pallas as pl
from jax.experimental.pallas import tpu as pltpu

N_Y = 4


def kernel(Q, K, V):
    b, s, h, d = Q.shape
    bh = b * h
    scale = d ** -0.5

    Qt = Q.transpose(0, 2, 3, 1).reshape(bh, d, s) * scale
    Kt = K.transpose(0, 2, 3, 1).reshape(bh, d, s)
    Vt = V.transpose(0, 2, 3, 1).reshape(bh, d, s)

    def body(q_ref, k_ref, v_ref, out_ref, kfull, vfull, send_sems, recv_sems):
        my_x = lax.axis_index("x")
        my_y = lax.axis_index("y")
        my_z = lax.axis_index("z")
        left = (my_y - 1) % N_Y
        right = (my_y + 1) % N_Y

        barrier = pltpu.get_barrier_semaphore()
        for nbr in (left, right):
            pl.semaphore_signal(
                barrier, inc=1,
                device_id=(my_x, nbr, my_z),
                device_id_type=pl.DeviceIdType.MESH,
            )
        pl.semaphore_wait(barrier, 2)

        hh = bh // 2
        right_dev = (my_x, right, my_z)
        left_dev = (my_x, left, my_z)

        def rdma(tensor, stream, src, dst, dev):
            return pltpu.make_async_remote_copy(
                src_ref=src,
                dst_ref=dst,
                send_sem=send_sems.at[tensor, stream],
                recv_sem=recv_sems.at[tensor, stream],
                device_id=dev,
                device_id_type=pl.DeviceIdType.MESH,
            )

        k_r0 = rdma(0, 0, k_ref, kfull.at[0], right_dev)
        v_r0 = rdma(1, 0, v_ref, vfull.at[0], right_dev)
        k_l0 = rdma(0, 1, k_ref, kfull.at[1], left_dev)
        v_l0 = rdma(1, 1, v_ref, vfull.at[1], left_dev)
        k_r0.start()
        v_r0.start()
        k_l0.start()
        v_l0.start()

        k_r0.wait()
        v_r0.wait()
        k_r1 = rdma(0, 2, kfull.at[0, pl.ds(0, hh)],
                    kfull.at[2, pl.ds(0, hh)], right_dev)
        v_r1 = rdma(1, 2, vfull.at[0, pl.ds(0, hh)],
                    vfull.at[2, pl.ds(0, hh)], right_dev)
        k_r1.start()
        v_r1.start()

        k_l0.wait()
        v_l0.wait()
        k_l1 = rdma(0, 3, kfull.at[1, pl.ds(hh, hh)],
                    kfull.at[2, pl.ds(hh, hh)], left_dev)
        v_l1 = rdma(1, 3, vfull.at[1, pl.ds(hh, hh)],
                    vfull.at[2, pl.ds(hh, hh)], left_dev)
        k_l1.start()
        v_l1.start()

        k_r1.wait()
        v_r1.wait()
        k_l1.wait()
        v_l1.wait()

        def head_body(i, carry):
            q = q_ref[i]
            k_chunks = [k_ref] + [kfull.at[t] for t in range(N_Y - 1)]
            v_chunks = [v_ref] + [vfull.at[t] for t in range(N_Y - 1)]
            s_blocks = []
            for kc in k_chunks:
                s_blocks.append(
                    lax.dot_general(
                        q, kc[i],
                        (((0,), (0,)), ((), ())),
                        preferred_element_type=jnp.float32,
                    )
                )
            S = jnp.concatenate(s_blocks, axis=1)
            m = jnp.max(S, axis=1, keepdims=True)
            P = jnp.exp(S - m)
            denom = jnp.sum(P, axis=1)[None, :]
            acc = jnp.zeros((d, s), jnp.float32)
            for c, vc in enumerate(v_chunks):
                acc = acc + lax.dot_general(
                    vc[i], P[:, c * s:(c + 1) * s],
                    (((1,), (1,)), ((), ())),
                    preferred_element_type=jnp.float32,
                )
            out_ref[i] = acc / denom
            return carry

        lax.fori_loop(0, bh, head_body, 0)

    out = pl.pallas_call(
        body,
        out_shape=jax.ShapeDtypeStruct((bh, d, s), jnp.float32),
        in_specs=[pl.BlockSpec(memory_space=pltpu.VMEM)] * 3,
        out_specs=pl.BlockSpec(memory_space=pltpu.VMEM),
        scratch_shapes=[
            pltpu.VMEM((N_Y - 1, bh, d, s), jnp.float32),
            pltpu.VMEM((N_Y - 1, bh, d, s), jnp.float32),
            pltpu.SemaphoreType.DMA((2, 4)),
            pltpu.SemaphoreType.DMA((2, 4)),
        ],
        compiler_params=pltpu.CompilerParams(collective_id=0),
    )(Qt, Kt, Vt)

    return out.reshape(b, h, d, s).transpose(0, 3, 1, 2)


# device time: 253717 ns/iter; 1.3372x vs baseline; 1.3295x over previous
import jax
import jax.numpy as jnp
from jax import lax
from jax.experimental import pallas as pl
from jax.experimental.pallas import tpu as pltpu

N_Y = 4
N_STEP = N_Y - 1

S_R = 0
S_L = 1
S_XR = 2
S_XL = 3


def kernel(Q, K, V):
    b, s, h, d = Q.shape
    bh = b * h
    hh = bh // 2
    scale = d ** -0.5

    Qt = Q.transpose(0, 2, 3, 1).reshape(bh, d, s) * scale
    Kt = K.transpose(0, 2, 3, 1).reshape(2, hh, d, s)
    Vt = V.transpose(0, 2, 3, 1).reshape(2, hh, d, s)
    KVt = jnp.stack([Kt, Vt], axis=1)

    def body(q_ref, kv_ref, out_ref, kvbuf, send_sems, recv_sems):
        my_x = lax.axis_index("x")
        my_y = lax.axis_index("y")
        my_z = lax.axis_index("z")
        mh = my_x
        oh = 1 - my_x
        twin_dev = (1 - my_x, my_y, my_z)
        here = (my_x, my_y, my_z)
        has_left = my_y > 0
        has_right = my_y < N_Y - 1

        left_dev = (my_x, jnp.maximum(my_y - 1, 0), my_z)
        right_dev = (my_x, jnp.minimum(my_y + 1, N_Y - 1), my_z)

        barrier = pltpu.get_barrier_semaphore()
        for dev in (twin_dev, left_dev, right_dev):
            pl.semaphore_signal(
                barrier, inc=1, device_id=dev,
                device_id_type=pl.DeviceIdType.MESH,
            )
        pl.semaphore_wait(barrier, 3)

        def mk(stream, t, src, dst, dev):
            return pltpu.make_async_remote_copy(
                src_ref=src,
                dst_ref=dst,
                send_sem=send_sems.at[stream, t],
                recv_sem=recv_sems.at[stream, t],
                device_id=dev,
                device_id_type=pl.DeviceIdType.MESH,
            )

        for t in range(N_STEP):
            @pl.when(has_right & (my_y - t >= 0))
            def _(t=t):
                c = my_y - t
                src = kv_ref.at[mh] if t == 0 else kvbuf.at[c, mh]
                mk(S_R, t, src, kvbuf.at[c, mh], right_dev).start()

            @pl.when(has_left & (my_y + t <= N_Y - 1))
            def _(t=t):
                c = my_y + t
                src = kv_ref.at[mh] if t == 0 else kvbuf.at[c - 1, mh]
                mk(S_L, t, src, kvbuf.at[c - 1, mh], left_dev).start()

            @pl.when(my_y - 1 - t >= 0)
            def _(t=t):
                c = my_y - 1 - t
                mk(S_R, t, kvbuf.at[c, mh], kvbuf.at[c, mh],
                   here).wait_recv()

            @pl.when(my_y - 1 - t >= 0)
            def _(t=t):
                c = my_y - 1 - t
                mk(S_XR, t, kvbuf.at[c, mh], kvbuf.at[c, mh],
                   twin_dev).start()

            @pl.when(my_y + 1 + t <= N_Y - 1)
            def _(t=t):
                c = my_y + 1 + t
                mk(S_L, t, kvbuf.at[c - 1, mh], kvbuf.at[c - 1, mh],
                   here).wait_recv()

            @pl.when(my_y + 1 + t <= N_Y - 1)
            def _(t=t):
                c = my_y + 1 + t
                mk(S_XL, t, kvbuf.at[c - 1, mh], kvbuf.at[c - 1, mh],
                   twin_dev).start()

        for t in range(N_STEP):
            @pl.when(my_y - 1 - t >= 0)
            def _(t=t):
                c = my_y - 1 - t
                mk(S_XR, t, kvbuf.at[c, oh], kvbuf.at[c, oh],
                   here).wait_recv()

            @pl.when(my_y + 1 + t <= N_Y - 1)
            def _(t=t):
                c = my_y + 1 + t
                mk(S_XL, t, kvbuf.at[c - 1, oh], kvbuf.at[c - 1, oh],
                   here).wait_recv()

        for t in range(N_STEP):
            @pl.when(has_right & (my_y - t >= 0))
            def _(t=t):
                c = my_y - t
                src = kv_ref.at[mh] if t == 0 else kvbuf.at[c, mh]
                mk(S_R, t, src, kvbuf.at[c, mh], right_dev).wait_send()

            @pl.when(has_left & (my_y + t <= N_Y - 1))
            def _(t=t):
                c = my_y + t
                src = kv_ref.at[mh] if t == 0 else kvbuf.at[c - 1, mh]
                mk(S_L, t, src, kvbuf.at[c - 1, mh], left_dev).wait_send()

            @pl.when(my_y - 1 - t >= 0)
            def _(t=t):
                c = my_y - 1 - t
                mk(S_XR, t, kvbuf.at[c, mh], kvbuf.at[c, mh],
                   twin_dev).wait_send()

            @pl.when(my_y + 1 + t <= N_Y - 1)
            def _(t=t):
                c = my_y + 1 + t
                mk(S_XL, t, kvbuf.at[c - 1, mh], kvbuf.at[c - 1, mh],
                   twin_dev).wait_send()

        def head_body(i, carry):
            q = q_ref[i]
            ih, ir = i // hh, i % hh
            k_blocks = [kv_ref.at[ih, 0, ir]] + [
                kvbuf.at[sl, ih, 0, ir] for sl in range(N_Y - 1)
            ]
            v_blocks = [kv_ref.at[ih, 1, ir]] + [
                kvbuf.at[sl, ih, 1, ir] for sl in range(N_Y - 1)
            ]
            s_blocks = []
            for kc in k_blocks:
                s_blocks.append(
                    lax.dot_general(
                        q, kc[...],
                        (((0,), (0,)), ((), ())),
                        preferred_element_type=jnp.float32,
                    )
                )
            S = jnp.concatenate(s_blocks, axis=1)
            m = jnp.max(S, axis=1, keepdims=True)
            P = jnp.exp(S - m)
            denom = jnp.sum(P, axis=1)[None, :]
            acc = jnp.zeros((d, s), jnp.float32)
            for c, vc in enumerate(v_blocks):
                acc = acc + lax.dot_general(
                    vc[...], P[:, c * s:(c + 1) * s],
                    (((1,), (1,)), ((), ())),
                    preferred_element_type=jnp.float32,
                )
            out_ref[i] = acc / denom
            return carry

        lax.fori_loop(0, bh, head_body, 0)

    out = pl.pallas_call(
        body,
        out_shape=jax.ShapeDtypeStruct((bh, d, s), jnp.float32),
        in_specs=[pl.BlockSpec(memory_space=pltpu.VMEM)] * 2,
        out_specs=pl.BlockSpec(memory_space=pltpu.VMEM),
        scratch_shapes=[
            pltpu.VMEM((N_Y - 1, 2, 2, hh, d, s), jnp.float32),
            pltpu.SemaphoreType.DMA((4, N_STEP)),
            pltpu.SemaphoreType.DMA((4, N_STEP)),
        ],
        compiler_params=pltpu.CompilerParams(collective_id=0),
    )(Qt, KVt)

    return out.reshape(b, h, d, s).transpose(0, 3, 1, 2)
